# baseline (speedup 1.0000x reference)
import functools

import jax
import jax.numpy as jnp
from jax import lax
from jax.experimental import pallas as pl
from jax.experimental.pallas import tpu as pltpu
from jax.experimental.pallas import tpu_sc as plsc

D_MODEL = 64
SCALE = 8.0

NUM_WORKERS = 32
IDX_COLS = 128
K = 4


def kernel(input, table):
    batch, hist = input.shape
    total = batch * hist
    assert total % (NUM_WORKERS * IDX_COLS * K) == 0
    idx2d = input.reshape(total // IDX_COLS, IDX_COLS).astype(jnp.int32)
    n_rows = idx2d.shape[0]
    rows_per_w = n_rows // NUM_WORKERS
    n_chunks = rows_per_w // K

    mesh = plsc.VectorSubcoreMesh(core_axis_name="c", subcore_axis_name="s")

    @functools.partial(
        pl.kernel,
        mesh=mesh,
        out_type=jax.ShapeDtypeStruct((n_rows, IDX_COLS, D_MODEL), jnp.float32),
        scratch_types=[
            pltpu.VMEM((K, IDX_COLS), jnp.int32),
            pltpu.VMEM((K, IDX_COLS, D_MODEL), jnp.float32),
            pltpu.SemaphoreType.DMA,
        ],
        compiler_params=pltpu.CompilerParams(use_tc_tiling_on_sc=False),
    )
    def emb(idx_hbm, table_hbm, out_hbm, idx_v, rows_v, sem):
        wid = lax.axis_index("s") * 2 + lax.axis_index("c")
        row0 = wid * rows_per_w

        def chunk(g, carry):
            rbase = row0 + g * K
            pltpu.sync_copy(idx_hbm.at[pl.ds(rbase, K)], idx_v)
            cps = [
                pltpu.async_copy(table_hbm.at[idx_v.at[j]], rows_v.at[j], sem)
                for j in range(K)
            ]
            for cp in cps:
                cp.wait()

            def scale_row(i, c2):
                for j in range(K):
                    for q in range(D_MODEL // 16):
                        sl = (j, i, pl.ds(q * 16, 16))
                        rows_v[sl] = rows_v[sl] * SCALE
                return c2

            lax.fori_loop(0, IDX_COLS, scale_row, 0)
            pltpu.sync_copy(rows_v, out_hbm.at[pl.ds(rbase, K)])
            return carry

        lax.fori_loop(0, n_chunks, chunk, 0)

    out = emb(idx2d, table)
    return out.reshape(batch, hist, D_MODEL)


# speedup vs baseline: 1.1454x; 1.1454x over previous
import functools

import jax
import jax.numpy as jnp
from jax import lax
from jax.experimental import pallas as pl
from jax.experimental.pallas import tpu as pltpu
from jax.experimental.pallas import tpu_sc as plsc

D_MODEL = 64
SCALE = 8.0

NUM_WORKERS = 32
IDX_COLS = 128
K = 5


def kernel(input, table):
    batch, hist = input.shape
    total = batch * hist
    assert total % (NUM_WORKERS * IDX_COLS * K * 2) == 0
    idx2d = input.reshape(total // IDX_COLS, IDX_COLS).astype(jnp.int32)
    n_rows = idx2d.shape[0]
    rows_per_w = n_rows // NUM_WORKERS
    n_chunks = rows_per_w // K
    n2 = n_chunks // 2

    mesh = plsc.VectorSubcoreMesh(core_axis_name="c", subcore_axis_name="s")

    @functools.partial(
        pl.kernel,
        mesh=mesh,
        out_type=jax.ShapeDtypeStruct((n_rows, IDX_COLS, D_MODEL), jnp.float32),
        scratch_types=[
            pltpu.VMEM((2, K, IDX_COLS), jnp.int32),
            pltpu.VMEM((2, K, IDX_COLS, D_MODEL), jnp.float32),
            pltpu.SemaphoreType.DMA,
            pltpu.SemaphoreType.DMA,
            pltpu.SemaphoreType.DMA,
            pltpu.SemaphoreType.DMA,
            pltpu.SemaphoreType.DMA,
            pltpu.SemaphoreType.DMA,
        ],
        compiler_params=pltpu.CompilerParams(use_tc_tiling_on_sc=False),
    )
    def emb(idx_hbm, table_hbm, out_hbm, idx_v, rows_v,
            isem0, isem1, gsem0, gsem1, ssem0, ssem1):
        isems = [isem0, isem1]
        gsems = [gsem0, gsem1]
        ssems = [ssem0, ssem1]
        wid = lax.axis_index("s") * 2 + lax.axis_index("c")
        row0 = wid * rows_per_w

        def fire_idx(g, b):
            pltpu.async_copy(idx_hbm.at[pl.ds(row0 + g * K, K)],
                             idx_v.at[b], isems[b])

        def wait_idx(b):
            pltpu.make_async_copy(idx_hbm.at[pl.ds(row0, K)],
                                  idx_v.at[b], isems[b]).wait()

        def fire_gathers(b):
            for j in range(K):
                pltpu.async_copy(table_hbm.at[idx_v.at[b, j]],
                                 rows_v.at[b, j], gsems[b])

        def wait_gathers(b):
            for j in range(K):
                pltpu.make_async_copy(table_hbm.at[idx_v.at[b, j]],
                                      rows_v.at[b, j], gsems[b]).wait()

        def fire_store(g, b):
            pltpu.async_copy(rows_v.at[b],
                             out_hbm.at[pl.ds(row0 + g * K, K)], ssems[b])

        def wait_store(b):
            pltpu.make_async_copy(rows_v.at[b],
                                  out_hbm.at[pl.ds(row0, K)], ssems[b]).wait()

        def scale(b):
            def srow(i, c):
                for j in range(K):
                    for q in range(D_MODEL // 16):
                        sl = (b, j, i, pl.ds(q * 16, 16))
                        rows_v[sl] = rows_v[sl] * SCALE
                return c
            lax.fori_loop(0, IDX_COLS, srow, 0)

        fire_idx(0, 0)
        fire_idx(1, 1)
        wait_idx(0)
        fire_gathers(0)

        def outer(p, c):
            wait_gathers(0)

            @pl.when(p < n2 - 1)
            def _():
                fire_idx(2 * p + 2, 0)

            @pl.when(p >= 1)
            def _():
                wait_store(1)

            wait_idx(1)
            fire_gathers(1)
            scale(0)
            fire_store(2 * p, 0)

            wait_gathers(1)

            @pl.when(p < n2 - 1)
            def _():
                fire_idx(2 * p + 3, 1)
                wait_store(0)
                wait_idx(0)
                fire_gathers(0)

            scale(1)
            fire_store(2 * p + 1, 1)
            return c

        lax.fori_loop(0, n2, outer, 0)
        wait_store(0)
        wait_store(1)

    out = emb(idx2d, table)
    return out.reshape(batch, hist, D_MODEL)


# speedup vs baseline: 1.2964x; 1.1318x over previous
import functools

import jax
import jax.numpy as jnp
from jax import lax
from jax.experimental import pallas as pl
from jax.experimental.pallas import tpu as pltpu
from jax.experimental.pallas import tpu_sc as plsc

D_MODEL = 64
SCALE = 8.0

NUM_WORKERS = 32
IDX_COLS = 128
K = 5
CHUNK = K * IDX_COLS


def kernel(input, table):
    batch, hist = input.shape
    total = batch * hist
    assert total % (NUM_WORKERS * CHUNK * 2) == 0
    idx2d = input.reshape(total // IDX_COLS, IDX_COLS).astype(jnp.int32)
    n_rows = idx2d.shape[0]
    rows_per_w = n_rows // NUM_WORKERS
    n_chunks = rows_per_w // K
    n2 = n_chunks // 2

    mesh = plsc.VectorSubcoreMesh(core_axis_name="c", subcore_axis_name="s")

    @functools.partial(
        pl.kernel,
        mesh=mesh,
        out_type=jax.ShapeDtypeStruct((total, D_MODEL), jnp.float32),
        scratch_types=[
            pltpu.VMEM((2, K, IDX_COLS), jnp.int32),
            pltpu.VMEM((2, CHUNK, D_MODEL), jnp.float32),
            pltpu.SemaphoreType.DMA,
            pltpu.SemaphoreType.DMA,
            pltpu.SemaphoreType.DMA,
            pltpu.SemaphoreType.DMA,
            pltpu.SemaphoreType.DMA,
            pltpu.SemaphoreType.DMA,
        ],
        compiler_params=pltpu.CompilerParams(use_tc_tiling_on_sc=False),
    )
    def emb(idx_hbm, table_hbm, out_hbm, idx_v, rows_v,
            isem0, isem1, gsem0, gsem1, ssem0, ssem1):
        isems = [isem0, isem1]
        gsems = [gsem0, gsem1]
        ssems = [ssem0, ssem1]
        wid = lax.axis_index("s") * 2 + lax.axis_index("c")
        row0 = wid * rows_per_w

        def fire_idx(g, b):
            pltpu.async_copy(idx_hbm.at[pl.ds(row0 + g * K, K)],
                             idx_v.at[b], isems[b])

        def wait_idx(b):
            pltpu.make_async_copy(idx_hbm.at[pl.ds(row0, K)],
                                  idx_v.at[b], isems[b]).wait()

        def fire_gathers(b):
            for j in range(K):
                pltpu.async_copy(table_hbm.at[idx_v.at[b, j]],
                                 rows_v.at[b, pl.ds(j * IDX_COLS, IDX_COLS)],
                                 gsems[b])

        def wait_gathers(b):
            for j in range(K):
                pltpu.make_async_copy(
                    table_hbm.at[idx_v.at[b, j]],
                    rows_v.at[b, pl.ds(j * IDX_COLS, IDX_COLS)],
                    gsems[b]).wait()

        def fire_store(g, b):
            pltpu.async_copy(rows_v.at[b],
                             out_hbm.at[pl.ds((row0 + g * K) * IDX_COLS, CHUNK)],
                             ssems[b])

        def wait_store(b):
            pltpu.make_async_copy(rows_v.at[b],
                                  out_hbm.at[pl.ds(row0 * IDX_COLS, CHUNK)],
                                  ssems[b]).wait()

        def scale(b):
            def srow(i, c):
                for q in range(D_MODEL // 16):
                    sl = (b, i, pl.ds(q * 16, 16))
                    rows_v[sl] = rows_v[sl] * SCALE
                return c
            lax.fori_loop(0, CHUNK, srow, 0)

        fire_idx(0, 0)
        fire_idx(1, 1)
        wait_idx(0)
        fire_gathers(0)

        def outer(p, c):
            wait_gathers(0)

            @pl.when(p < n2 - 1)
            def _():
                fire_idx(2 * p + 2, 0)

            @pl.when(p >= 1)
            def _():
                wait_store(1)

            wait_idx(1)
            fire_gathers(1)
            scale(0)
            fire_store(2 * p, 0)

            wait_gathers(1)

            @pl.when(p < n2 - 1)
            def _():
                fire_idx(2 * p + 3, 1)
                wait_store(0)
                wait_idx(0)
                fire_gathers(0)

            scale(1)
            fire_store(2 * p + 1, 1)
            return c

        lax.fori_loop(0, n2, outer, 0)
        wait_store(0)
        wait_store(1)

    out = emb(idx2d, table)
    return out.reshape(batch, hist, D_MODEL)
